# baseline (speedup 1.0000x reference)
import functools
import math

import jax
import jax.numpy as jnp
from jax import lax
from jax.experimental import pallas as pl
from jax.experimental.pallas import tpu as pltpu
from jax.experimental.pallas import tpu_sc as plsc

H, W = 480, 640
NMAP = 9
CELLS = 10000
CPAD = 10240
ROWS_PER_TILE = 30
ROWS_PER_CHUNK = 2
CHUNKS = ROWS_PER_TILE // ROWS_PER_CHUNK
VECS = W // 16
RED = NMAP * CPAD // 16
UNROLL = 3
_F = 320.0 / math.tan(math.radians(39.5))
C1 = 1.0 / (5.0 * _F)
BAND_LO, BAND_HI = 11, 35


def _floor(x):
    ti = x.astype(jnp.int32)
    tf = ti.astype(jnp.float32)
    corr = (x < tf)
    return ti - corr.astype(jnp.int32), tf - corr.astype(jnp.float32)


def _axis_weights(p, dim):
    i0, f0 = _floor(p)
    fr = p - f0
    i1 = i0 + 1
    m0 = (i0 > 0) & (i0 < dim)
    m1 = (i1 > 0) & (i1 < dim)
    w0 = (1.0 - fr) * m0.astype(jnp.float32)
    w1 = fr * m1.astype(jnp.float32)
    s0 = jnp.where(m0, i0, 0)
    s1 = jnp.where(m1, i1, 0)
    return (s0, w0), (s1, w1)


def _sc_body(obs_hbm, thr_hbm, stage_hbm, out_hbm,
             a0, a1, a2, a3, a4, a5, a6, a7, a8, buf, tbuf,
             rb0, rb1, racc, sem, semb):
    accs = [a0, a1, a2, a3, a4, a5, a6, a7, a8]
    g = lax.axis_index("c")
    s = lax.axis_index("s")
    r0 = s * ROWS_PER_TILE
    wid = g * 16 + s

    def _zbody(i, _):
        z = jnp.zeros((16,), jnp.float32)
        for a in accs:
            a[pl.ds(i * 16, 16)] = z
        return 0
    lax.fori_loop(0, CPAD // 16, _zbody, 0)

    pltpu.sync_copy(thr_hbm, tbuf)

    def _pipeline(group):
        if group == 0:
            src_planes = [3, 4, 5, 6, 7, 8, 9, 10]
        else:
            src_planes = [3, 11, 12, 13, 14, 15, 16, 17, 18, 19]
        nfeat = len(src_planes) - 1
        sems = [sem, semb]

        def _issue(row, par):
            for k, c in enumerate(src_planes):
                pltpu.async_copy(obs_hbm.at[pl.ds(c * (H * W) + row * W, W)],
                                 buf.at[par, k], sems[par])

        _issue(r0, 0)
        _issue(r0 + 1, 1)

        def _chunk(ci, _):
            for par in (0, 1):
                ri = ci * 2 + par
                r = r0 + ri
                for k, c in enumerate(src_planes):
                    pltpu.make_async_copy(
                        obs_hbm.at[pl.ds(c * (H * W) + r * W, W)],
                        buf.at[par, k], sems[par]).wait()
                gzn = 239.5 - r.astype(jnp.float32)

                @plsc.parallel_loop(0, VECS, step=1, unroll=UNROLL)
                def _vec(vi, par=par, gzn=gzn):
                    base = vi * 16
                    tv = tbuf[...]
                    iota_f = lax.iota(jnp.int32, 16).astype(jnp.float32)
                    dep = buf[par, 0, pl.ds(base, 16)]
                    dep = jnp.where(dep < tv, 10000.0, dep)
                    gxo = iota_f + (base.astype(jnp.float32) - 319.5)
                    px = gxo * dep * C1 + 50.0
                    py = dep * 0.2
                    pz = dep * (gzn * C1) + 25.6

                    xs = _axis_weights(px, 100)
                    ys = _axis_weights(py, 100)
                    (z0, wz0), (z1, wz1) = _axis_weights(pz, 80)
                    b0 = ((z0 >= BAND_LO) & (z0 < BAND_HI)).astype(jnp.float32)
                    b1 = ((z1 >= BAND_LO) & (z1 < BAND_HI)).astype(jnp.float32)
                    A = wz0 + wz1
                    B = wz0 * b0 + wz1 * b1

                    feats = [buf[par, k + 1, pl.ds(base, 16)]
                             for k in range(nfeat)]

                    (y0s, wy0), (y1s, wy1) = ys
                    yrows = [(y0s * 100, wy0), (y1s * 100, wy1)]
                    for (yrow, wy) in yrows:
                        for (xi, wx) in xs:
                            cell = yrow + xi
                            w = wx * wy
                            wB = w * B
                            if group == 0:
                                plsc.addupdate_scatter(accs[0], [cell], w * A)
                                plsc.addupdate_scatter(accs[1], [cell], wB)
                                for k in range(nfeat):
                                    plsc.addupdate_scatter(
                                        accs[2 + k], [cell], wB * feats[k])
                            else:
                                for k in range(nfeat):
                                    plsc.addupdate_scatter(
                                        accs[k], [cell], wB * feats[k])

                @pl.when(ri + 2 < ROWS_PER_TILE)
                def _():
                    _issue(r + 2, par)
            return 0
        lax.fori_loop(0, ROWS_PER_TILE // 2, _chunk, 0)

    pl.when(g == 0)(lambda: _pipeline(0))
    pl.when(g == 1)(lambda: _pipeline(1))

    for m in range(NMAP):
        pltpu.sync_copy(accs[m],
                        stage_hbm.at[pl.ds(wid * (NMAP * CPAD) + m * CPAD,
                                           CPAD)])
    plsc.subcore_barrier()

    off = s * RED
    gbase = g * 16 * (NMAP * CPAD)
    rbufs = [rb0, rb1]
    cps = [pltpu.async_copy(
               stage_hbm.at[pl.ds(gbase + j * (NMAP * CPAD) + off, RED)],
               rbufs[j % 2], sem)
           for j in range(2)]

    def _zred(i, _):
        racc[pl.ds(i * 16, 16)] = jnp.zeros((16,), jnp.float32)
        return 0
    lax.fori_loop(0, RED // 16, _zred, 0)

    for j in range(16):
        cps[j % 2].wait()
        rb = rbufs[j % 2]

        @plsc.parallel_loop(0, RED // 16, step=1, unroll=4)
        def _radd(i, rb=rb):
            ix = pl.ds(i * 16, 16)
            racc[ix] = racc[ix] + rb[ix]

        if j + 2 < 16:
            cps[j % 2] = pltpu.async_copy(
                stage_hbm.at[pl.ds(gbase + (j + 2) * (NMAP * CPAD) + off,
                                   RED)],
                rbufs[j % 2], sem)

    pltpu.sync_copy(racc, out_hbm.at[pl.ds(g * (NMAP * CPAD) + off, RED)])


_sc_splat = functools.partial(
    pl.kernel,
    mesh=plsc.VectorSubcoreMesh(core_axis_name="c", subcore_axis_name="s"),
    compiler_params=pltpu.CompilerParams(needs_layout_passes=False,
                                         use_tc_tiling_on_sc=False),
    out_type=[jax.ShapeDtypeStruct((32 * NMAP * CPAD,), jnp.float32),
              jax.ShapeDtypeStruct((2 * NMAP * CPAD,), jnp.float32)],
    scratch_types=(
        [pltpu.VMEM((CPAD,), jnp.float32) for _ in range(NMAP)]
        + [pltpu.VMEM((2, 10, W), jnp.float32),
           pltpu.VMEM((16,), jnp.float32),
           pltpu.VMEM((RED,), jnp.float32),
           pltpu.VMEM((RED,), jnp.float32),
           pltpu.VMEM((RED,), jnp.float32),
           pltpu.SemaphoreType.DMA,
           pltpu.SemaphoreType.DMA]
    ),
)(_sc_body)


def _finish_body(p_ref, po_ref, pls_ref, win_ref, pose_ref):
    p = p_ref[...][:, :, :CELLS]
    s0 = p[0]
    s1 = p[1]
    win = jnp.concatenate([
        jnp.clip(s0[1:2], 0.0, 1.0),
        jnp.clip(s0[0:1], 0.0, 1.0),
        jnp.zeros((2, CELLS), jnp.float32),
        jnp.clip(s0[2:9] * 0.2, 0.0, 1.0),
        jnp.clip(s1 * 0.2, 0.0, 1.0),
    ], axis=0)
    win_ref[...] = win

    po = po_ref[...]
    pls = pls_ref[...]
    d = 57.29577951308232
    t = pls[:, 2] / d
    st, ct = jnp.sin(t), jnp.cos(t)
    py = pls[:, 1] + po[:, 0] * st + po[:, 1] * ct
    px = pls[:, 0] + po[:, 0] * ct - po[:, 1] * st
    pt = pls[:, 2] + po[:, 2] * d
    pt = jnp.fmod(pt - 180.0, 360.0) + 180.0
    pt = jnp.fmod(pt + 180.0, 360.0) - 180.0
    pose_ref[...] = jnp.stack([px, py, pt], axis=1)


def _finish(partials, pose_obs, poses_last):
    return pl.pallas_call(
        _finish_body,
        out_shape=[jax.ShapeDtypeStruct((20, CELLS), jnp.float32),
                   jax.ShapeDtypeStruct((1, 3), jnp.float32)],
    )(partials, pose_obs, poses_last)


def kernel(obs, pose_obs, maps_last, poses_last, holding_state, holding_obj,
           holding_box, holding_obj_with_hole):
    del maps_last, holding_obj, holding_obj_with_hole
    obs = obs.astype(jnp.float32)
    in_state = (holding_state == 1) | (holding_state == 2)
    thresh = jnp.where(in_state,
                       jnp.where(holding_box != 0, 70.0, 50.0),
                       0.0).astype(jnp.float32)
    thr_vec = jnp.full((16,), thresh, jnp.float32)

    _, reduced = _sc_splat(obs.reshape(-1), thr_vec)
    partials = reduced.reshape(2, NMAP, CPAD)
    win, poses = _finish(partials, pose_obs.astype(jnp.float32),
                         poses_last.astype(jnp.float32))
    win = win.reshape(20, 100, 100)
    agent_view = jnp.zeros((1, 20, 480, 480), jnp.float32)
    agent_view = agent_view.at[0, :, 240:336, 190:290].set(win[:, 0:96, :])
    return agent_view, poses


# speedup vs baseline: 1.1755x; 1.1755x over previous
import functools
import math

import jax
import jax.numpy as jnp
from jax import lax
from jax.experimental import pallas as pl
from jax.experimental.pallas import tpu as pltpu
from jax.experimental.pallas import tpu_sc as plsc

H, W = 480, 640
NMAP = 9
CELLS = 10000
CPAD = 10240
ROWS_PER_TILE = 30
ROWS_PER_CHUNK = 2
CHUNKS = ROWS_PER_TILE // ROWS_PER_CHUNK
VECS = W // 16
RED = NMAP * CPAD // 16
UNROLL = 2
_F = 320.0 / math.tan(math.radians(39.5))
C1 = 1.0 / (5.0 * _F)
BAND_LO, BAND_HI = 11, 35


def _floor(x):
    ti = x.astype(jnp.int32)
    tf = ti.astype(jnp.float32)
    corr = (x < tf)
    return ti - corr.astype(jnp.int32), tf - corr.astype(jnp.float32)


def _axis_weights(p, dim):
    i0, f0 = _floor(p)
    fr = p - f0
    i1 = i0 + 1
    m0 = (i0 > 0) & (i0 < dim)
    m1 = (i1 > 0) & (i1 < dim)
    w0 = (1.0 - fr) * m0.astype(jnp.float32)
    w1 = fr * m1.astype(jnp.float32)
    s0 = jnp.where(m0, i0, 0)
    s1 = jnp.where(m1, i1, 0)
    return (s0, w0), (s1, w1)


def _sc_body(obs_hbm, thr_hbm, stage_hbm, out_hbm,
             a0, a1, a2, a3, a4, a5, a6, a7, a8, buf, tbuf,
             rb0, rb1, racc, sem, semb):
    accs = [a0, a1, a2, a3, a4, a5, a6, a7, a8]
    g = lax.axis_index("c")
    s = lax.axis_index("s")
    r0 = s * ROWS_PER_TILE
    wid = g * 16 + s

    def _zbody(i, _):
        z = jnp.zeros((16,), jnp.float32)
        for a in accs:
            a[pl.ds(i * 16, 16)] = z
        return 0
    lax.fori_loop(0, CPAD // 16, _zbody, 0)

    pltpu.sync_copy(thr_hbm, tbuf)

    def _pipeline(group):
        if group == 0:
            src_planes = [3, 4, 5, 6, 7, 8, 9, 10]
        else:
            src_planes = [3, 11, 12, 13, 14, 15, 16, 17, 18, 19]
        nfeat = len(src_planes) - 1
        sems = [sem, semb]

        def _issue(row, par):
            for k, c in enumerate(src_planes):
                pltpu.async_copy(obs_hbm.at[pl.ds(c * (H * W) + row * W, W)],
                                 buf.at[par, k], sems[par])

        _issue(r0, 0)
        _issue(r0 + 1, 1)

        def _chunk(ci, _):
            for par in (0, 1):
                ri = ci * 2 + par
                r = r0 + ri
                for k, c in enumerate(src_planes):
                    pltpu.make_async_copy(
                        obs_hbm.at[pl.ds(c * (H * W) + r * W, W)],
                        buf.at[par, k], sems[par]).wait()
                gzn = 239.5 - r.astype(jnp.float32)

                @plsc.parallel_loop(0, VECS, step=1, unroll=UNROLL)
                def _vec(vi, par=par, gzn=gzn):
                    base = vi * 16
                    tv = tbuf[...]
                    iota_f = lax.iota(jnp.int32, 16).astype(jnp.float32)
                    dep = buf[par, 0, pl.ds(base, 16)]
                    dep = jnp.where(dep < tv, 10000.0, dep)
                    gxo = iota_f + (base.astype(jnp.float32) - 319.5)
                    px = gxo * dep * C1 + 50.0
                    py = dep * 0.2
                    pz = dep * (gzn * C1) + 25.6

                    xs = _axis_weights(px, 100)
                    ys = _axis_weights(py, 100)
                    (z0, wz0), (z1, wz1) = _axis_weights(pz, 80)
                    b0 = ((z0 >= BAND_LO) & (z0 < BAND_HI)).astype(jnp.float32)
                    b1 = ((z1 >= BAND_LO) & (z1 < BAND_HI)).astype(jnp.float32)
                    A = wz0 + wz1
                    B = wz0 * b0 + wz1 * b1

                    feats = [buf[par, k + 1, pl.ds(base, 16)]
                             for k in range(nfeat)]

                    (y0s, wy0), (y1s, wy1) = ys
                    yrows = [(y0s * 100, wy0), (y1s * 100, wy1)]
                    for (yrow, wy) in yrows:
                        for (xi, wx) in xs:
                            cell = yrow + xi
                            w = wx * wy
                            wB = w * B
                            if group == 0:
                                plsc.addupdate_scatter(accs[0], [cell], w * A)
                                plsc.addupdate_scatter(accs[1], [cell], wB)
                                for k in range(nfeat):
                                    plsc.addupdate_scatter(
                                        accs[2 + k], [cell], wB * feats[k])
                            else:
                                for k in range(nfeat):
                                    plsc.addupdate_scatter(
                                        accs[k], [cell], wB * feats[k])

                @pl.when(ri + 2 < ROWS_PER_TILE)
                def _():
                    _issue(r + 2, par)
            return 0
        lax.fori_loop(0, ROWS_PER_TILE // 2, _chunk, 0)

    pl.when(g == 0)(lambda: _pipeline(0))
    pl.when(g == 1)(lambda: _pipeline(1))

    for m in range(NMAP):
        pltpu.sync_copy(accs[m],
                        stage_hbm.at[pl.ds(wid * (NMAP * CPAD) + m * CPAD,
                                           CPAD)])
    plsc.subcore_barrier()

    off = s * RED
    gbase = g * 16 * (NMAP * CPAD)
    rbufs = [rb0, rb1]
    cps = [pltpu.async_copy(
               stage_hbm.at[pl.ds(gbase + j * (NMAP * CPAD) + off, RED)],
               rbufs[j % 2], sem)
           for j in range(2)]

    def _zred(i, _):
        racc[pl.ds(i * 16, 16)] = jnp.zeros((16,), jnp.float32)
        return 0
    lax.fori_loop(0, RED // 16, _zred, 0)

    for j in range(16):
        cps[j % 2].wait()
        rb = rbufs[j % 2]

        @plsc.parallel_loop(0, RED // 16, step=1, unroll=4)
        def _radd(i, rb=rb):
            ix = pl.ds(i * 16, 16)
            racc[ix] = racc[ix] + rb[ix]

        if j + 2 < 16:
            cps[j % 2] = pltpu.async_copy(
                stage_hbm.at[pl.ds(gbase + (j + 2) * (NMAP * CPAD) + off,
                                   RED)],
                rbufs[j % 2], sem)

    pltpu.sync_copy(racc, out_hbm.at[pl.ds(g * (NMAP * CPAD) + off, RED)])


_sc_splat = functools.partial(
    pl.kernel,
    mesh=plsc.VectorSubcoreMesh(core_axis_name="c", subcore_axis_name="s"),
    compiler_params=pltpu.CompilerParams(needs_layout_passes=False,
                                         use_tc_tiling_on_sc=False),
    out_type=[jax.ShapeDtypeStruct((32 * NMAP * CPAD,), jnp.float32),
              jax.ShapeDtypeStruct((2 * NMAP * CPAD,), jnp.float32)],
    scratch_types=(
        [pltpu.VMEM((CPAD,), jnp.float32) for _ in range(NMAP)]
        + [pltpu.VMEM((2, 10, W), jnp.float32),
           pltpu.VMEM((16,), jnp.float32),
           pltpu.VMEM((RED,), jnp.float32),
           pltpu.VMEM((RED,), jnp.float32),
           pltpu.VMEM((RED,), jnp.float32),
           pltpu.SemaphoreType.DMA,
           pltpu.SemaphoreType.DMA]
    ),
)(_sc_body)


def _finish_body(p_ref, po_ref, pls_ref, win_ref, pose_ref):
    p = p_ref[...][:, :, :CELLS]
    s0 = p[0]
    s1 = p[1]
    win = jnp.concatenate([
        jnp.clip(s0[1:2], 0.0, 1.0),
        jnp.clip(s0[0:1], 0.0, 1.0),
        jnp.zeros((2, CELLS), jnp.float32),
        jnp.clip(s0[2:9] * 0.2, 0.0, 1.0),
        jnp.clip(s1 * 0.2, 0.0, 1.0),
    ], axis=0)
    win_ref[...] = win

    po = po_ref[...]
    pls = pls_ref[...]
    d = 57.29577951308232
    t = pls[:, 2] / d
    st, ct = jnp.sin(t), jnp.cos(t)
    py = pls[:, 1] + po[:, 0] * st + po[:, 1] * ct
    px = pls[:, 0] + po[:, 0] * ct - po[:, 1] * st
    pt = pls[:, 2] + po[:, 2] * d
    pt = jnp.fmod(pt - 180.0, 360.0) + 180.0
    pt = jnp.fmod(pt + 180.0, 360.0) - 180.0
    pose_ref[...] = jnp.stack([px, py, pt], axis=1)


def _finish(partials, pose_obs, poses_last):
    return pl.pallas_call(
        _finish_body,
        out_shape=[jax.ShapeDtypeStruct((20, CELLS), jnp.float32),
                   jax.ShapeDtypeStruct((1, 3), jnp.float32)],
    )(partials, pose_obs, poses_last)


def kernel(obs, pose_obs, maps_last, poses_last, holding_state, holding_obj,
           holding_box, holding_obj_with_hole):
    del maps_last, holding_obj, holding_obj_with_hole
    obs = obs.astype(jnp.float32)
    in_state = (holding_state == 1) | (holding_state == 2)
    thresh = jnp.where(in_state,
                       jnp.where(holding_box != 0, 70.0, 50.0),
                       0.0).astype(jnp.float32)
    thr_vec = jnp.full((16,), thresh, jnp.float32)

    _, reduced = _sc_splat(obs.reshape(-1), thr_vec)
    partials = reduced.reshape(2, NMAP, CPAD)
    win, poses = _finish(partials, pose_obs.astype(jnp.float32),
                         poses_last.astype(jnp.float32))
    win = win.reshape(20, 100, 100)
    agent_view = jnp.zeros((1, 20, 480, 480), jnp.float32)
    agent_view = agent_view.at[0, :, 240:336, 190:290].set(win[:, 0:96, :])
    return agent_view, poses


# speedup vs baseline: 1.1815x; 1.0051x over previous
import functools
import math

import jax
import jax.numpy as jnp
from jax import lax
from jax.experimental import pallas as pl
from jax.experimental.pallas import tpu as pltpu
from jax.experimental.pallas import tpu_sc as plsc

H, W = 480, 640
NMAP = 9
CELLS = 10000
CPAD = 10240
ROWS_PER_TILE = 30
ROWS_PER_CHUNK = 2
CHUNKS = ROWS_PER_TILE // ROWS_PER_CHUNK
VECS = W // 16
RED = NMAP * CPAD // 16
UNROLL = 2
_F = 320.0 / math.tan(math.radians(39.5))
C1 = 1.0 / (5.0 * _F)
BAND_LO, BAND_HI = 11, 35


def _floor(x):
    ti = x.astype(jnp.int32)
    tf = ti.astype(jnp.float32)
    corr = (x < tf)
    return ti - corr.astype(jnp.int32), tf - corr.astype(jnp.float32)


def _axis_weights(p, dim):
    i0, f0 = _floor(p)
    fr = p - f0
    i1 = i0 + 1
    m0 = (i0 > 0) & (i0 < dim)
    m1 = (i1 > 0) & (i1 < dim)
    w0 = (1.0 - fr) * m0.astype(jnp.float32)
    w1 = fr * m1.astype(jnp.float32)
    s0 = jnp.where(m0, i0, 0)
    s1 = jnp.where(m1, i1, 0)
    return (s0, w0), (s1, w1)


def _sc_body(obs_hbm, thr_hbm, stage_hbm, out_hbm,
             a0, a1, a2, a3, a4, a5, a6, a7, a8, buf, tbuf,
             rb0, rb1, racc, sem, semb):
    accs = [a0, a1, a2, a3, a4, a5, a6, a7, a8]
    g = lax.axis_index("c")
    s = lax.axis_index("s")
    r0 = s * ROWS_PER_TILE
    wid = g * 16 + s

    def _zbody(i, _):
        z = jnp.zeros((16,), jnp.float32)
        for a in accs:
            a[pl.ds(i * 16, 16)] = z
        return 0
    lax.fori_loop(0, CPAD // 16, _zbody, 0)

    pltpu.sync_copy(thr_hbm, tbuf)

    def _pipeline(group):
        if group == 0:
            src_planes = [3, 4, 5, 6, 7, 8, 9, 10]
        else:
            src_planes = [3, 11, 12, 13, 14, 15, 16, 17, 18, 19]
        nfeat = len(src_planes) - 1
        sems = [sem, semb]

        def _issue(row, par):
            for k, c in enumerate(src_planes):
                pltpu.async_copy(obs_hbm.at[pl.ds(c * (H * W) + row * W, W)],
                                 buf.at[par, k], sems[par])

        _issue(r0, 0)
        _issue(r0 + 1, 1)

        def _chunk(ci, _):
            for par in (0, 1):
                ri = ci * 2 + par
                r = r0 + ri
                for k, c in enumerate(src_planes):
                    pltpu.make_async_copy(
                        obs_hbm.at[pl.ds(c * (H * W) + r * W, W)],
                        buf.at[par, k], sems[par]).wait()
                gzn = 239.5 - r.astype(jnp.float32)
                tv = tbuf[...]
                iota_f = lax.iota(jnp.int32, 16).astype(jnp.float32)

                @plsc.parallel_loop(0, VECS, step=1, unroll=UNROLL)
                def _vec(vi, par=par, gzn=gzn, tv=tv, iota_f=iota_f):
                    base = vi * 16
                    dep = buf[par, 0, pl.ds(base, 16)]
                    dep = jnp.where(dep < tv, 10000.0, dep)
                    gxo = iota_f + (base.astype(jnp.float32) - 319.5)
                    px = gxo * dep * C1 + 50.0
                    py = dep * 0.2
                    pz = dep * (gzn * C1) + 25.6

                    xs = _axis_weights(px, 100)
                    ys = _axis_weights(py, 100)
                    (z0, wz0), (z1, wz1) = _axis_weights(pz, 80)
                    b0 = ((z0 >= BAND_LO) & (z0 < BAND_HI)).astype(jnp.float32)
                    b1 = ((z1 >= BAND_LO) & (z1 < BAND_HI)).astype(jnp.float32)
                    A = wz0 + wz1
                    B = wz0 * b0 + wz1 * b1

                    feats = [buf[par, k + 1, pl.ds(base, 16)]
                             for k in range(nfeat)]

                    (y0s, wy0), (y1s, wy1) = ys
                    yrows = [(y0s * 100, wy0), (y1s * 100, wy1)]
                    for (yrow, wy) in yrows:
                        for (xi, wx) in xs:
                            cell = yrow + xi
                            w = wx * wy
                            wB = w * B
                            if group == 0:
                                plsc.addupdate_scatter(accs[0], [cell], w * A)
                                plsc.addupdate_scatter(accs[1], [cell], wB)
                                for k in range(nfeat):
                                    plsc.addupdate_scatter(
                                        accs[2 + k], [cell], wB * feats[k])
                            else:
                                for k in range(nfeat):
                                    plsc.addupdate_scatter(
                                        accs[k], [cell], wB * feats[k])

                @pl.when(ri + 2 < ROWS_PER_TILE)
                def _issue_next(par=par, r=r):
                    _issue(r + 2, par)
            return 0
        lax.fori_loop(0, ROWS_PER_TILE // 2, _chunk, 0)

    pl.when(g == 0)(lambda: _pipeline(0))
    pl.when(g == 1)(lambda: _pipeline(1))

    for m in range(NMAP):
        pltpu.sync_copy(accs[m],
                        stage_hbm.at[pl.ds(wid * (NMAP * CPAD) + m * CPAD,
                                           CPAD)])
    plsc.subcore_barrier()

    off = s * RED
    gbase = g * 16 * (NMAP * CPAD)
    rbufs = [rb0, rb1]
    cps = [pltpu.async_copy(
               stage_hbm.at[pl.ds(gbase + j * (NMAP * CPAD) + off, RED)],
               rbufs[j % 2], sem)
           for j in range(2)]

    def _zred(i, _):
        racc[pl.ds(i * 16, 16)] = jnp.zeros((16,), jnp.float32)
        return 0
    lax.fori_loop(0, RED // 16, _zred, 0)

    for j in range(16):
        cps[j % 2].wait()
        rb = rbufs[j % 2]

        @plsc.parallel_loop(0, RED // 16, step=1, unroll=4)
        def _radd(i, rb=rb):
            ix = pl.ds(i * 16, 16)
            racc[ix] = racc[ix] + rb[ix]

        if j + 2 < 16:
            cps[j % 2] = pltpu.async_copy(
                stage_hbm.at[pl.ds(gbase + (j + 2) * (NMAP * CPAD) + off,
                                   RED)],
                rbufs[j % 2], sem)

    pltpu.sync_copy(racc, out_hbm.at[pl.ds(g * (NMAP * CPAD) + off, RED)])


_sc_splat = functools.partial(
    pl.kernel,
    mesh=plsc.VectorSubcoreMesh(core_axis_name="c", subcore_axis_name="s"),
    compiler_params=pltpu.CompilerParams(needs_layout_passes=False,
                                         use_tc_tiling_on_sc=False),
    out_type=[jax.ShapeDtypeStruct((32 * NMAP * CPAD,), jnp.float32),
              jax.ShapeDtypeStruct((2 * NMAP * CPAD,), jnp.float32)],
    scratch_types=(
        [pltpu.VMEM((CPAD,), jnp.float32) for _ in range(NMAP)]
        + [pltpu.VMEM((2, 10, W), jnp.float32),
           pltpu.VMEM((16,), jnp.float32),
           pltpu.VMEM((RED,), jnp.float32),
           pltpu.VMEM((RED,), jnp.float32),
           pltpu.VMEM((RED,), jnp.float32),
           pltpu.SemaphoreType.DMA,
           pltpu.SemaphoreType.DMA]
    ),
)(_sc_body)


def _finish_body(p_ref, po_ref, pls_ref, win_ref, pose_ref):
    p = p_ref[...][:, :, :CELLS]
    s0 = p[0]
    s1 = p[1]
    win = jnp.concatenate([
        jnp.clip(s0[1:2], 0.0, 1.0),
        jnp.clip(s0[0:1], 0.0, 1.0),
        jnp.zeros((2, CELLS), jnp.float32),
        jnp.clip(s0[2:9] * 0.2, 0.0, 1.0),
        jnp.clip(s1 * 0.2, 0.0, 1.0),
    ], axis=0)
    win_ref[...] = win

    po = po_ref[...]
    pls = pls_ref[...]
    d = 57.29577951308232
    t = pls[:, 2] / d
    st, ct = jnp.sin(t), jnp.cos(t)
    py = pls[:, 1] + po[:, 0] * st + po[:, 1] * ct
    px = pls[:, 0] + po[:, 0] * ct - po[:, 1] * st
    pt = pls[:, 2] + po[:, 2] * d
    pt = jnp.fmod(pt - 180.0, 360.0) + 180.0
    pt = jnp.fmod(pt + 180.0, 360.0) - 180.0
    pose_ref[...] = jnp.stack([px, py, pt], axis=1)


def _finish(partials, pose_obs, poses_last):
    return pl.pallas_call(
        _finish_body,
        out_shape=[jax.ShapeDtypeStruct((20, CELLS), jnp.float32),
                   jax.ShapeDtypeStruct((1, 3), jnp.float32)],
    )(partials, pose_obs, poses_last)


def kernel(obs, pose_obs, maps_last, poses_last, holding_state, holding_obj,
           holding_box, holding_obj_with_hole):
    del maps_last, holding_obj, holding_obj_with_hole
    obs = obs.astype(jnp.float32)
    in_state = (holding_state == 1) | (holding_state == 2)
    thresh = jnp.where(in_state,
                       jnp.where(holding_box != 0, 70.0, 50.0),
                       0.0).astype(jnp.float32)
    thr_vec = jnp.full((16,), thresh, jnp.float32)

    _, reduced = _sc_splat(obs.reshape(-1), thr_vec)
    partials = reduced.reshape(2, NMAP, CPAD)
    win, poses = _finish(partials, pose_obs.astype(jnp.float32),
                         poses_last.astype(jnp.float32))
    win = win.reshape(20, 100, 100)
    agent_view = jnp.zeros((1, 20, 480, 480), jnp.float32)
    agent_view = agent_view.at[0, :, 240:336, 190:290].set(win[:, 0:96, :])
    return agent_view, poses
